# baseline (speedup 1.0000x reference)
import functools

import jax
import jax.numpy as jnp
from jax import lax
from jax.experimental import pallas as pl
from jax.experimental.pallas import tpu as pltpu
from jax.experimental.pallas import tpu_sc as plsc

N_NODES = 10000
N_EDGES = 160000
D = 256

NC = 2
NS = 16
NW = NC * NS
EPW = N_EDGES // NW
CH = 64
NCHUNK = 80
DEPTH = 4
TAIL_OFF = EPW - CH


def _chunk_off(c):
    return jnp.minimum(c * CH, TAIL_OFF)


def _dot_body(src_hbm, dst_hbm, h_hbm, out_hbm,
              idx_src_v, idx_dst_v, rows, outs, m_v, gsems, osems):
    wid = lax.axis_index("s") * NC + lax.axis_index("c")
    base = wid * EPW
    pltpu.sync_copy(src_hbm.at[pl.ds(base, EPW)], idx_src_v)
    pltpu.sync_copy(dst_hbm.at[pl.ds(base, EPW)], idx_dst_v)

    lane = lax.iota(jnp.int32, 16)
    cols = [jnp.full((16,), k, jnp.int32) for k in range(16)]

    def issue(c, p):
        off = _chunk_off(c)
        rs, rd = rows[p]
        pltpu.async_copy(h_hbm.at[idx_src_v.at[pl.ds(off, CH)]], rs, gsems[p])
        pltpu.async_copy(h_hbm.at[idx_dst_v.at[pl.ds(off, CH)]], rd, gsems[p])

    def wait_rows(p):
        rs, rd = rows[p]
        dummy = h_hbm.at[pl.ds(0, CH), :]
        pltpu.make_async_copy(dummy, rs, gsems[p]).wait()
        pltpu.make_async_copy(dummy, rd, gsems[p]).wait()

    def wait_out(p):
        pltpu.make_async_copy(outs[p], out_hbm.at[pl.ds(0, CH)],
                              osems[p]).wait()

    def compute_chunk(p):
        rs, rd = rows[p]
        ob = outs[p]

        def group_body(g, _):
            for ee in range(16):
                e = g * 16 + ee
                s = rs[e, pl.ds(0, 32)]
                d = rd[e, pl.ds(0, 32)]
                sa, sb = plsc.unpack(s, format=plsc.PackFormat.INTERLEAVED)
                da, db = plsc.unpack(d, format=plsc.PackFormat.INTERLEAVED)
                acc0 = sa * da
                acc1 = sb * db
                for k in range(1, D // 32):
                    s = rs[e, pl.ds(k * 32, 32)]
                    d = rd[e, pl.ds(k * 32, 32)]
                    sa, sb = plsc.unpack(s, format=plsc.PackFormat.INTERLEAVED)
                    da, db = plsc.unpack(d, format=plsc.PackFormat.INTERLEAVED)
                    acc0 = acc0 + sa * da
                    acc1 = acc1 + sb * db
                m_v[ee, pl.ds(0, 16)] = acc0 + acc1
            tot = plsc.load_gather(m_v, [lane, cols[0]])
            for k in range(1, 16):
                tot = tot + plsc.load_gather(m_v, [lane, cols[k]])
            ob[pl.ds(g * 16, 16)] = tot
            return _

        lax.fori_loop(0, CH // 16, group_body, None)

    for p in range(DEPTH):
        issue(p, p)

    def quad_body(j4, _):
        for p in range(DEPTH):
            c = DEPTH * j4 + p
            wait_rows(p)

            @pl.when(j4 > 0)
            def _wo():
                wait_out(p)

            compute_chunk(p)
            pltpu.async_copy(outs[p],
                             out_hbm.at[pl.ds(base + _chunk_off(c), CH)],
                             osems[p])

            @pl.when(j4 < NCHUNK // DEPTH - 1)
            def _is():
                issue(c + DEPTH, p)

        return _

    lax.fori_loop(0, NCHUNK // DEPTH, quad_body, None)
    for p in range(DEPTH):
        wait_out(p)


@functools.partial(jax.jit, static_argnames=())
def kernel(edge_index, h):
    src = edge_index[0].astype(jnp.int32)
    dst = edge_index[1].astype(jnp.int32)
    hb = h.astype(jnp.bfloat16)

    def body(src_hbm, dst_hbm, h_hbm, out_hbm,
             idx_src_v, idx_dst_v,
             rs0, rd0, rs1, rd1, rs2, rd2, rs3, rd3,
             o0, o1, o2, o3, m_v,
             g0, g1, g2, g3, s0, s1, s2, s3):
        _dot_body(src_hbm, dst_hbm, h_hbm, out_hbm,
                  idx_src_v, idx_dst_v,
                  [(rs0, rd0), (rs1, rd1), (rs2, rd2), (rs3, rd3)],
                  [o0, o1, o2, o3], m_v,
                  [g0, g1, g2, g3], [s0, s1, s2, s3])

    mesh = plsc.VectorSubcoreMesh(core_axis_name="c", subcore_axis_name="s")
    return pl.kernel(
        body,
        out_type=jax.ShapeDtypeStruct((N_EDGES,), jnp.float32),
        mesh=mesh,
        compiler_params=pltpu.CompilerParams(use_tc_tiling_on_sc=False,
                                             needs_layout_passes=False),
        scratch_types=(
            [pltpu.VMEM((EPW,), jnp.int32)] * 2
            + [pltpu.VMEM((CH, D), jnp.bfloat16)] * (2 * DEPTH)
            + [pltpu.VMEM((CH,), jnp.float32)] * DEPTH
            + [pltpu.VMEM((16, 17), jnp.float32)]
            + [pltpu.SemaphoreType.DMA] * (2 * DEPTH)
        ),
    )(src, dst, hb)
